# baseline (speedup 1.0000x reference)
import functools

import jax
import jax.numpy as jnp
import numpy as np
from jax import lax
from jax.experimental import pallas as pl
from jax.experimental.pallas import tpu as pltpu
from jax.experimental.pallas import tpu_sc as plsc

H = W = 512
NUM = H * W
NPAIR = int(NUM * 0.15)
NTILE = 32

TP_CHUNK = 1024
TP_NCHUNK = NUM // NTILE // TP_CHUNK

PCH = 2048
NPCH = 4
NCH = NTILE * NPCH

_COMPILER_PARAMS = pltpu.CompilerParams(
    needs_layout_passes=False, use_tc_tiling_on_sc=False)
_COMPILER_PARAMS_TILED = pltpu.CompilerParams(
    needs_layout_passes=False, use_tc_tiling_on_sc=True)
_MESH = plsc.VectorSubcoreMesh(core_axis_name="c", subcore_axis_name="s")


def _pair_partition():
    rng = np.random.RandomState(0)
    p1 = rng.choice(NUM, NPAIR, replace=True)
    rng.shuffle(p1)
    p2 = rng.choice(NUM, NPAIR, replace=True)
    rng.shuffle(p2)
    counts = np.zeros(NCH, np.int64)
    anchor = np.empty(NPAIR, np.int64)
    other = np.empty(NPAIR, np.int64)
    c1 = p1 // PCH
    c2 = p2 // PCH
    for i in range(NPAIR):
        if counts[c1[i]] <= counts[c2[i]]:
            a, o = p1[i], p2[i]
        else:
            a, o = p2[i], p1[i]
        anchor[i] = a
        other[i] = o
        counts[a // PCH] += 1
    pc = int(-(-counts.max() // 128) * 128)
    i1 = np.zeros((NCH, pc), np.int32)
    p2g = np.zeros((NCH, pc), np.int32)
    for c in range(NCH):
        p2g[c, :] = c * PCH
    fill = np.zeros(NCH, np.int64)
    for i in range(NPAIR):
        c = anchor[i] // PCH
        j = fill[c]
        fill[c] += 1
        i1[c, j] = anchor[i] - c * PCH
        p2g[c, j] = other[i]
    return (i1.reshape(NTILE, NPCH, pc),
            p2g.reshape(NTILE, NPCH, pc // 128, 128), pc)


_I1_NP, _P2_NP, PC = _pair_partition()
NDESC = PC // 128


def _tp_body(gt_hbm, pr_hbm, t_hbm, slab_a, slab_b, tch_a, tch_b,
             sem_in_a, sem_in_b, sem_out_a, sem_out_b):
    c = lax.axis_index("c")
    s = lax.axis_index("s")
    wid = s * 2 + c
    base_reg = wid * TP_NCHUNK
    iota = lax.iota(jnp.int32, 16)
    lane_hi = lax.shift_right_logical(iota, 3)
    lane_lo16 = (iota & 7) * 16
    lane_idx = [lane_lo16 + k for k in range(16)]
    dy_consts = [jnp.full((16,), dy, jnp.int32) for dy in range(8)]
    trow_consts = [xg * 2 + lane_hi for xg in range(8)]

    def issue_slabs(ri, slab, sem):
        y0 = lax.shift_right_logical(ri, 2) * 8
        x0 = (ri & 3) * 128
        for b in range(8):
            pltpu.async_copy(
                gt_hbm.at[b, 0, pl.ds(y0, 8), pl.ds(x0, 128)],
                slab.at[b], sem)
            pltpu.async_copy(
                pr_hbm.at[b, 0, pl.ds(y0, 8), pl.ds(x0, 128)],
                slab.at[b + 8], sem)

    def wait_slabs(slab, sem):
        for k in range(16):
            pltpu.make_async_copy(
                gt_hbm.at[0, 0, pl.ds(0, 8), pl.ds(0, 128)],
                slab.at[k], sem).wait()

    def compute(ri, slab, tch, sem_out):
        for g in range(64):
            dy = g >> 3
            xg = g & 7
            for k in range(16):
                v = slab[k, dy, pl.ds(xg * 16, 16)]
                plsc.store_scatter(
                    tch, [dy_consts[dy], trow_consts[xg], lane_idx[k]], v)
        y0 = lax.shift_right_logical(ri, 2) * 8
        x0r = (ri & 3) * 16
        for dy in range(8):
            tr0 = (y0 + dy) * 64 + x0r
            pltpu.async_copy(tch.at[dy], t_hbm.at[pl.ds(tr0, 16), :],
                             sem_out)

    def wait_out(tch, sem):
        for dy in range(8):
            pltpu.make_async_copy(t_hbm.at[pl.ds(0, 16), :], tch.at[dy],
                                  sem).wait()

    issue_slabs(base_reg, slab_a, sem_in_a)

    def loop(i, carry):
        r0 = base_reg + 2 * i
        issue_slabs(r0 + 1, slab_b, sem_in_b)
        wait_slabs(slab_a, sem_in_a)

        @pl.when(i >= 1)
        def _():
            wait_out(tch_a, sem_out_a)

        compute(r0, slab_a, tch_a, sem_out_a)

        @pl.when(i < (TP_NCHUNK // 2) - 1)
        def _():
            issue_slabs(r0 + 2, slab_a, sem_in_a)

        wait_slabs(slab_b, sem_in_b)

        @pl.when(i >= 1)
        def _():
            wait_out(tch_b, sem_out_b)

        compute(r0 + 1, slab_b, tch_b, sem_out_b)
        return carry

    lax.fori_loop(0, TP_NCHUNK // 2, loop, 0)
    wait_out(tch_a, sem_out_a)
    wait_out(tch_b, sem_out_b)


_sc_transpose = functools.partial(
    pl.kernel,
    mesh=_MESH,
    compiler_params=_COMPILER_PARAMS_TILED,
    out_type=jax.ShapeDtypeStruct((NUM // 8, 128), jnp.float32),
    scratch_types=[
        pltpu.VMEM((16, 8, 128), jnp.float32),
        pltpu.VMEM((16, 8, 128), jnp.float32),
        pltpu.VMEM((8, 16, 128), jnp.float32),
        pltpu.VMEM((8, 16, 128), jnp.float32),
        pltpu.SemaphoreType.DMA,
        pltpu.SemaphoreType.DMA,
        pltpu.SemaphoreType.DMA,
        pltpu.SemaphoreType.DMA,
    ],
)(_tp_body)


def _pair_compute(ci, i1_v, slab, pbuf, acc, iota):
    for g in range(PC // 16):
        b1 = i1_v[ci, pl.ds(g * 16, 16)]
        rowi = g * 16 + iota
        ls = []
        for b in range(8):
            cb = jnp.full((16,), b, jnp.int32)
            cq = jnp.full((16,), b + 8, jnp.int32)
            g1 = plsc.load_gather(slab, [b1, cb])
            g2 = plsc.load_gather(pbuf, [rowi, cb])
            q1 = plsc.load_gather(slab, [b1, cq])
            q2 = plsc.load_gather(pbuf, [rowi, cq])
            gd = g1 - g2
            pd = q1 - q2
            ls.append(jnp.where(gd - gd == 0.0, jnp.abs(gd - pd), 0.0))
        tot = ls[0]
        for b in range(1, 8):
            tot = tot + ls[b]
        lo = [jnp.minimum(ls[2 * i], ls[2 * i + 1]) for i in range(4)]
        hi = [jnp.maximum(ls[2 * i], ls[2 * i + 1]) for i in range(4)]
        m1l = jnp.minimum(lo[0], lo[1])
        m1h = jnp.minimum(jnp.maximum(lo[0], lo[1]),
                          jnp.minimum(hi[0], hi[1]))
        m2l = jnp.minimum(lo[2], lo[3])
        m2h = jnp.minimum(jnp.maximum(lo[2], lo[3]),
                          jnp.minimum(hi[2], hi[3]))
        f1 = jnp.minimum(m1l, m2l)
        f2 = jnp.minimum(jnp.maximum(m1l, m2l), jnp.minimum(m1h, m2h))
        acc = acc + (tot - f1 - f2)
    return acc


def _sc_body(t_hbm, i1_hbm, p2_hbm, out_hbm,
             i1_v, p2i_v, slab_a, slab_b, pb_a, pb_b, row_v, slab16, shared,
             ssa, ssb, spa, spb):
    c = lax.axis_index("c")
    s = lax.axis_index("s")
    wid = s * 2 + c

    pltpu.sync_copy(i1_hbm.at[wid], i1_v)
    pltpu.sync_copy(p2_hbm.at[wid], p2i_v)

    iota = lax.iota(jnp.int32, 16)

    def issue(ci, slab, pbuf, ssem, psem):
        pix0 = wid * (NPCH * PCH) + ci * PCH
        pltpu.async_copy(t_hbm.at[pl.ds(pix0, PCH), :], slab, ssem)
        for d in range(NDESC):
            pltpu.async_copy(t_hbm.at[p2i_v.at[ci, d]],
                             pbuf.at[pl.ds(d * 128, 128), :], psem)

    def wait(slab, pbuf, ssem, psem):
        pltpu.make_async_copy(t_hbm.at[pl.ds(0, PCH), :], slab, ssem).wait()
        for d in range(NDESC):
            pltpu.make_async_copy(t_hbm.at[p2i_v.at[0, 0]],
                                  pbuf.at[pl.ds(d * 128, 128), :],
                                  psem).wait()

    issue(0, slab_a, pb_a, ssa, spa)

    def loop(i, acc):
        c0 = 2 * i
        issue(c0 + 1, slab_b, pb_b, ssb, spb)
        wait(slab_a, pb_a, ssa, spa)
        acc = _pair_compute(c0, i1_v, slab_a, pb_a, acc, iota)

        @pl.when(i < (NPCH // 2) - 1)
        def _():
            issue(c0 + 2, slab_a, pb_a, ssa, spa)

        wait(slab_b, pb_b, ssb, spb)
        acc = _pair_compute(c0 + 1, i1_v, slab_b, pb_b, acc, iota)
        return acc

    acc = lax.fori_loop(0, NPCH // 2, loop, jnp.zeros((16,), jnp.float32))

    row_v[0, :] = acc
    pltpu.sync_copy(row_v, shared.at[pl.ds(s, 1), :])
    plsc.subcore_barrier()

    @pl.when(s == 0)
    def _():
        pltpu.sync_copy(shared, slab16)
        tot = slab16[0, :]
        for r in range(1, 16):
            tot = tot + slab16[r, :]
        row_v[0, :] = tot
        pltpu.sync_copy(row_v, out_hbm.at[c])


_sc_pairloss = functools.partial(
    pl.kernel,
    mesh=_MESH,
    compiler_params=_COMPILER_PARAMS,
    out_type=jax.ShapeDtypeStruct((2, 1, 16), jnp.float32),
    scratch_types=[
        pltpu.VMEM((NPCH, PC), jnp.int32),
        pltpu.VMEM((NPCH, NDESC, 128), jnp.int32),
        pltpu.VMEM((PCH, 16), jnp.float32),
        pltpu.VMEM((PCH, 16), jnp.float32),
        pltpu.VMEM((PC, 16), jnp.float32),
        pltpu.VMEM((PC, 16), jnp.float32),
        pltpu.VMEM((1, 16), jnp.float32),
        pltpu.VMEM((16, 16), jnp.float32),
        pltpu.VMEM_SHARED((16, 16), jnp.float32),
        pltpu.SemaphoreType.DMA,
        pltpu.SemaphoreType.DMA,
        pltpu.SemaphoreType.DMA,
        pltpu.SemaphoreType.DMA,
    ],
)(_sc_body)


def kernel(gt_depth, pred_depth):
    table = _sc_transpose(gt_depth, pred_depth)
    i1 = jnp.asarray(_I1_NP)
    p2 = jnp.asarray(_P2_NP)
    parts = _sc_pairloss(table.reshape(NUM, 16), i1, p2)
    return jnp.sum(parts) * np.float32(1.0 / (6 * NPAIR))


# speedup vs baseline: 1.1990x; 1.1990x over previous
import functools

import jax
import jax.numpy as jnp
import numpy as np
from jax import lax
from jax.experimental import pallas as pl
from jax.experimental.pallas import tpu as pltpu
from jax.experimental.pallas import tpu_sc as plsc

H = W = 512
NUM = H * W
NPAIR = int(NUM * 0.15)
NTILE = 32

TP_CHUNK = 1024
TP_NCHUNK = NUM // NTILE // TP_CHUNK

PCH = 2048
NPCH = 4
NCH = NTILE * NPCH

_COMPILER_PARAMS = pltpu.CompilerParams(
    needs_layout_passes=False, use_tc_tiling_on_sc=False)
_COMPILER_PARAMS_TILED = pltpu.CompilerParams(
    needs_layout_passes=False, use_tc_tiling_on_sc=True)
_MESH = plsc.VectorSubcoreMesh(core_axis_name="c", subcore_axis_name="s")


def _pair_partition():
    rng = np.random.RandomState(0)
    p1 = rng.choice(NUM, NPAIR, replace=True)
    rng.shuffle(p1)
    p2 = rng.choice(NUM, NPAIR, replace=True)
    rng.shuffle(p2)
    counts = np.zeros(NCH, np.int64)
    anchor = np.empty(NPAIR, np.int64)
    other = np.empty(NPAIR, np.int64)
    c1 = p1 // PCH
    c2 = p2 // PCH
    for i in range(NPAIR):
        if counts[c1[i]] <= counts[c2[i]]:
            a, o = p1[i], p2[i]
        else:
            a, o = p2[i], p1[i]
        anchor[i] = a
        other[i] = o
        counts[a // PCH] += 1
    pc = int(-(-counts.max() // 128) * 128)
    i1 = np.zeros((NCH, pc), np.int32)
    p2g = np.zeros((NCH, pc), np.int32)
    for c in range(NCH):
        p2g[c, :] = c * PCH
    fill = np.zeros(NCH, np.int64)
    for i in range(NPAIR):
        c = anchor[i] // PCH
        j = fill[c]
        fill[c] += 1
        i1[c, j] = anchor[i] - c * PCH
        p2g[c, j] = other[i]
    return (i1.reshape(NTILE, NPCH, pc),
            p2g.reshape(NTILE, NPCH, pc // 128, 128), pc)


_I1_NP, _P2_NP, PC = _pair_partition()
NDESC = PC // 128


def _tp_body(gt_hbm, pr_hbm, t_hbm, slab_a, slab_b, tch_a, tch_b,
             sem_in_a, sem_in_b, sem_out_a, sem_out_b):
    c = lax.axis_index("c")
    s = lax.axis_index("s")
    wid = s * 2 + c
    base_reg = wid * TP_NCHUNK
    iota = lax.iota(jnp.int32, 16)
    lane_hi = lax.shift_right_logical(iota, 3)
    lane_lo16 = (iota & 7) * 16
    lane_idx = [lane_lo16 + k for k in range(16)]

    def issue_slabs(ri, slab, sem):
        y0 = lax.shift_right_logical(ri, 2) * 8
        x0 = (ri & 3) * 128
        for b in range(8):
            pltpu.async_copy(
                gt_hbm.at[b, 0, pl.ds(y0, 8), pl.ds(x0, 128)],
                slab.at[b], sem)
            pltpu.async_copy(
                pr_hbm.at[b, 0, pl.ds(y0, 8), pl.ds(x0, 128)],
                slab.at[b + 8], sem)

    def wait_slabs(slab, sem):
        for k in range(16):
            pltpu.make_async_copy(
                gt_hbm.at[0, 0, pl.ds(0, 8), pl.ds(0, 128)],
                slab.at[k], sem).wait()

    def compute(ri, slab, tch, sem_out):
        def group(g, _):
            dy = lax.shift_right_logical(g, 3)
            xg = g & 7
            dyv = jnp.full((16,), 0, jnp.int32) + dy
            trow = xg * 2 + lane_hi
            for k in range(16):
                v = slab[k, dy, pl.ds(xg * 16, 16)]
                plsc.store_scatter(tch, [dyv, trow, lane_idx[k]], v)
            return 0
        lax.fori_loop(0, 64, group, 0)
        y0 = lax.shift_right_logical(ri, 2) * 8
        x0r = (ri & 3) * 16
        for dy in range(8):
            tr0 = (y0 + dy) * 64 + x0r
            pltpu.async_copy(tch.at[dy], t_hbm.at[pl.ds(tr0, 16), :],
                             sem_out)

    def wait_out(tch, sem):
        for dy in range(8):
            pltpu.make_async_copy(t_hbm.at[pl.ds(0, 16), :], tch.at[dy],
                                  sem).wait()

    issue_slabs(base_reg, slab_a, sem_in_a)

    def loop(i, carry):
        r0 = base_reg + 2 * i
        issue_slabs(r0 + 1, slab_b, sem_in_b)
        wait_slabs(slab_a, sem_in_a)

        @pl.when(i >= 1)
        def _():
            wait_out(tch_a, sem_out_a)

        compute(r0, slab_a, tch_a, sem_out_a)

        @pl.when(i < (TP_NCHUNK // 2) - 1)
        def _():
            issue_slabs(r0 + 2, slab_a, sem_in_a)

        wait_slabs(slab_b, sem_in_b)

        @pl.when(i >= 1)
        def _():
            wait_out(tch_b, sem_out_b)

        compute(r0 + 1, slab_b, tch_b, sem_out_b)
        return carry

    lax.fori_loop(0, TP_NCHUNK // 2, loop, 0)
    wait_out(tch_a, sem_out_a)
    wait_out(tch_b, sem_out_b)


_sc_transpose = functools.partial(
    pl.kernel,
    mesh=_MESH,
    compiler_params=_COMPILER_PARAMS_TILED,
    out_type=jax.ShapeDtypeStruct((NUM // 8, 128), jnp.float32),
    scratch_types=[
        pltpu.VMEM((16, 8, 128), jnp.float32),
        pltpu.VMEM((16, 8, 128), jnp.float32),
        pltpu.VMEM((8, 16, 128), jnp.float32),
        pltpu.VMEM((8, 16, 128), jnp.float32),
        pltpu.SemaphoreType.DMA,
        pltpu.SemaphoreType.DMA,
        pltpu.SemaphoreType.DMA,
        pltpu.SemaphoreType.DMA,
    ],
)(_tp_body)


def _pair_compute(ci, i1_v, slab, pbuf, acc, iota):
    for g in range(PC // 16):
        b1 = i1_v[ci, pl.ds(g * 16, 16)]
        rowi = g * 16 + iota
        ls = []
        for b in range(8):
            cb = jnp.full((16,), b, jnp.int32)
            cq = jnp.full((16,), b + 8, jnp.int32)
            g1 = plsc.load_gather(slab, [b1, cb])
            g2 = plsc.load_gather(pbuf, [rowi, cb])
            q1 = plsc.load_gather(slab, [b1, cq])
            q2 = plsc.load_gather(pbuf, [rowi, cq])
            gd = g1 - g2
            pd = q1 - q2
            ls.append(jnp.where(gd - gd == 0.0, jnp.abs(gd - pd), 0.0))
        tot = ls[0]
        for b in range(1, 8):
            tot = tot + ls[b]
        lo = [jnp.minimum(ls[2 * i], ls[2 * i + 1]) for i in range(4)]
        hi = [jnp.maximum(ls[2 * i], ls[2 * i + 1]) for i in range(4)]
        m1l = jnp.minimum(lo[0], lo[1])
        m1h = jnp.minimum(jnp.maximum(lo[0], lo[1]),
                          jnp.minimum(hi[0], hi[1]))
        m2l = jnp.minimum(lo[2], lo[3])
        m2h = jnp.minimum(jnp.maximum(lo[2], lo[3]),
                          jnp.minimum(hi[2], hi[3]))
        f1 = jnp.minimum(m1l, m2l)
        f2 = jnp.minimum(jnp.maximum(m1l, m2l), jnp.minimum(m1h, m2h))
        acc = acc + (tot - f1 - f2)
    return acc


def _sc_body(t_hbm, i1_hbm, p2_hbm, out_hbm,
             i1_v, p2i_v, slab_a, slab_b, pb_a, pb_b, row_v, slab16, shared,
             ssa, ssb, spa, spb):
    c = lax.axis_index("c")
    s = lax.axis_index("s")
    wid = s * 2 + c

    pltpu.sync_copy(i1_hbm.at[wid], i1_v)
    pltpu.sync_copy(p2_hbm.at[wid], p2i_v)

    iota = lax.iota(jnp.int32, 16)

    def issue(ci, slab, pbuf, ssem, psem):
        pix0 = wid * (NPCH * PCH) + ci * PCH
        pltpu.async_copy(t_hbm.at[pl.ds(pix0, PCH), :], slab, ssem)
        for d in range(NDESC):
            pltpu.async_copy(t_hbm.at[p2i_v.at[ci, d]],
                             pbuf.at[pl.ds(d * 128, 128), :], psem)

    def wait(slab, pbuf, ssem, psem):
        pltpu.make_async_copy(t_hbm.at[pl.ds(0, PCH), :], slab, ssem).wait()
        for d in range(NDESC):
            pltpu.make_async_copy(t_hbm.at[p2i_v.at[0, 0]],
                                  pbuf.at[pl.ds(d * 128, 128), :],
                                  psem).wait()

    issue(0, slab_a, pb_a, ssa, spa)

    def loop(i, acc):
        c0 = 2 * i
        issue(c0 + 1, slab_b, pb_b, ssb, spb)
        wait(slab_a, pb_a, ssa, spa)
        acc = _pair_compute(c0, i1_v, slab_a, pb_a, acc, iota)

        @pl.when(i < (NPCH // 2) - 1)
        def _():
            issue(c0 + 2, slab_a, pb_a, ssa, spa)

        wait(slab_b, pb_b, ssb, spb)
        acc = _pair_compute(c0 + 1, i1_v, slab_b, pb_b, acc, iota)
        return acc

    acc = lax.fori_loop(0, NPCH // 2, loop, jnp.zeros((16,), jnp.float32))

    row_v[0, :] = acc
    pltpu.sync_copy(row_v, shared.at[pl.ds(s, 1), :])
    plsc.subcore_barrier()

    @pl.when(s == 0)
    def _():
        pltpu.sync_copy(shared, slab16)
        tot = slab16[0, :]
        for r in range(1, 16):
            tot = tot + slab16[r, :]
        row_v[0, :] = tot
        pltpu.sync_copy(row_v, out_hbm.at[c])


_sc_pairloss = functools.partial(
    pl.kernel,
    mesh=_MESH,
    compiler_params=_COMPILER_PARAMS,
    out_type=jax.ShapeDtypeStruct((2, 1, 16), jnp.float32),
    scratch_types=[
        pltpu.VMEM((NPCH, PC), jnp.int32),
        pltpu.VMEM((NPCH, NDESC, 128), jnp.int32),
        pltpu.VMEM((PCH, 16), jnp.float32),
        pltpu.VMEM((PCH, 16), jnp.float32),
        pltpu.VMEM((PC, 16), jnp.float32),
        pltpu.VMEM((PC, 16), jnp.float32),
        pltpu.VMEM((1, 16), jnp.float32),
        pltpu.VMEM((16, 16), jnp.float32),
        pltpu.VMEM_SHARED((16, 16), jnp.float32),
        pltpu.SemaphoreType.DMA,
        pltpu.SemaphoreType.DMA,
        pltpu.SemaphoreType.DMA,
        pltpu.SemaphoreType.DMA,
    ],
)(_sc_body)


def kernel(gt_depth, pred_depth):
    table = _sc_transpose(gt_depth, pred_depth)
    i1 = jnp.asarray(_I1_NP)
    p2 = jnp.asarray(_P2_NP)
    parts = _sc_pairloss(table.reshape(NUM, 16), i1, p2)
    return jnp.sum(parts) * np.float32(1.0 / (6 * NPAIR))


# speedup vs baseline: 1.2232x; 1.0202x over previous
import functools

import jax
import jax.numpy as jnp
import numpy as np
from jax import lax
from jax.experimental import pallas as pl
from jax.experimental.pallas import tpu as pltpu
from jax.experimental.pallas import tpu_sc as plsc

H = W = 512
NUM = H * W
NPAIR = int(NUM * 0.15)
NTILE = 32

TP_CHUNK = 1024
TP_NCHUNK = NUM // NTILE // TP_CHUNK

PCH = 2048
NPCH = 4
NCH = NTILE * NPCH

_COMPILER_PARAMS = pltpu.CompilerParams(
    needs_layout_passes=False, use_tc_tiling_on_sc=False)
_COMPILER_PARAMS_TILED = pltpu.CompilerParams(
    needs_layout_passes=False, use_tc_tiling_on_sc=True)
_MESH = plsc.VectorSubcoreMesh(core_axis_name="c", subcore_axis_name="s")


def _pair_partition():
    rng = np.random.RandomState(0)
    p1 = rng.choice(NUM, NPAIR, replace=True)
    rng.shuffle(p1)
    p2 = rng.choice(NUM, NPAIR, replace=True)
    rng.shuffle(p2)
    counts = np.zeros(NCH, np.int64)
    anchor = np.empty(NPAIR, np.int64)
    other = np.empty(NPAIR, np.int64)
    c1 = p1 // PCH
    c2 = p2 // PCH
    for i in range(NPAIR):
        if counts[c1[i]] <= counts[c2[i]]:
            a, o = p1[i], p2[i]
        else:
            a, o = p2[i], p1[i]
        anchor[i] = a
        other[i] = o
        counts[a // PCH] += 1
    pc = int(-(-counts.max() // 128) * 128)
    i1 = np.zeros((NCH, pc), np.int32)
    p2g = np.zeros((NCH, pc), np.int32)
    for c in range(NCH):
        i1[c, :] = c * PCH
        p2g[c, :] = c * PCH
    fill = np.zeros(NCH, np.int64)
    for i in range(NPAIR):
        c = anchor[i] // PCH
        j = fill[c]
        fill[c] += 1
        i1[c, j] = anchor[i]
        p2g[c, j] = other[i]
    for c in range(NCH):
        order = np.argsort(i1[c], kind="stable")
        i1[c] = i1[c][order]
        p2g[c] = p2g[c][order]
    return (i1.reshape(NTILE, NPCH, pc // 128, 128),
            p2g.reshape(NTILE, NPCH, pc // 128, 128), pc)


_I1_NP, _P2_NP, PC = _pair_partition()
NDESC = PC // 128


def _tp_body(gt_hbm, pr_hbm, t_hbm, slab_a, slab_b, tch_a, tch_b,
             sem_in_a, sem_in_b, sem_out_a, sem_out_b):
    c = lax.axis_index("c")
    s = lax.axis_index("s")
    wid = s * 2 + c
    base_reg = wid * TP_NCHUNK
    iota = lax.iota(jnp.int32, 16)
    lane_hi = lax.shift_right_logical(iota, 3)
    lane_lo16 = (iota & 7) * 16
    lane_idx = [lane_lo16 + k for k in range(16)]

    def issue_slabs(ri, slab, sem):
        y0 = lax.shift_right_logical(ri, 2) * 8
        x0 = (ri & 3) * 128
        for b in range(8):
            pltpu.async_copy(
                gt_hbm.at[b, 0, pl.ds(y0, 8), pl.ds(x0, 128)],
                slab.at[b], sem)
            pltpu.async_copy(
                pr_hbm.at[b, 0, pl.ds(y0, 8), pl.ds(x0, 128)],
                slab.at[b + 8], sem)

    def wait_slabs(slab, sem):
        for k in range(16):
            pltpu.make_async_copy(
                gt_hbm.at[0, 0, pl.ds(0, 8), pl.ds(0, 128)],
                slab.at[k], sem).wait()

    def compute(ri, slab, tch, sem_out):
        def group(g, _):
            dy = lax.shift_right_logical(g, 3)
            xg = g & 7
            dyv = jnp.full((16,), 0, jnp.int32) + dy
            trow = xg * 2 + lane_hi
            for k in range(16):
                v = slab[k, dy, pl.ds(xg * 16, 16)]
                plsc.store_scatter(tch, [dyv, trow, lane_idx[k]], v)
            return 0
        lax.fori_loop(0, 64, group, 0)
        y0 = lax.shift_right_logical(ri, 2) * 8
        x0r = (ri & 3) * 16
        for dy in range(8):
            tr0 = (y0 + dy) * 64 + x0r
            pltpu.async_copy(tch.at[dy], t_hbm.at[pl.ds(tr0, 16), :],
                             sem_out)

    def wait_out(tch, sem):
        for dy in range(8):
            pltpu.make_async_copy(t_hbm.at[pl.ds(0, 16), :], tch.at[dy],
                                  sem).wait()

    issue_slabs(base_reg, slab_a, sem_in_a)

    def loop(i, carry):
        r0 = base_reg + 2 * i
        issue_slabs(r0 + 1, slab_b, sem_in_b)
        wait_slabs(slab_a, sem_in_a)

        @pl.when(i >= 1)
        def _():
            wait_out(tch_a, sem_out_a)

        compute(r0, slab_a, tch_a, sem_out_a)

        @pl.when(i < (TP_NCHUNK // 2) - 1)
        def _():
            issue_slabs(r0 + 2, slab_a, sem_in_a)

        wait_slabs(slab_b, sem_in_b)

        @pl.when(i >= 1)
        def _():
            wait_out(tch_b, sem_out_b)

        compute(r0 + 1, slab_b, tch_b, sem_out_b)
        return carry

    lax.fori_loop(0, TP_NCHUNK // 2, loop, 0)
    wait_out(tch_a, sem_out_a)
    wait_out(tch_b, sem_out_b)


_sc_transpose = functools.partial(
    pl.kernel,
    mesh=_MESH,
    compiler_params=_COMPILER_PARAMS_TILED,
    out_type=jax.ShapeDtypeStruct((NUM // 8, 128), jnp.float32),
    scratch_types=[
        pltpu.VMEM((16, 8, 128), jnp.float32),
        pltpu.VMEM((16, 8, 128), jnp.float32),
        pltpu.VMEM((8, 16, 128), jnp.float32),
        pltpu.VMEM((8, 16, 128), jnp.float32),
        pltpu.SemaphoreType.DMA,
        pltpu.SemaphoreType.DMA,
        pltpu.SemaphoreType.DMA,
        pltpu.SemaphoreType.DMA,
    ],
)(_tp_body)


def _pair_compute(abuf, pbuf, acc, iota):
    for g in range(PC // 16):
        rowi = g * 16 + iota
        ls = []
        for b in range(8):
            cb = jnp.full((16,), b, jnp.int32)
            cq = jnp.full((16,), b + 8, jnp.int32)
            g1 = plsc.load_gather(abuf, [rowi, cb])
            g2 = plsc.load_gather(pbuf, [rowi, cb])
            q1 = plsc.load_gather(abuf, [rowi, cq])
            q2 = plsc.load_gather(pbuf, [rowi, cq])
            gd = g1 - g2
            pd = q1 - q2
            ls.append(jnp.where(gd - gd == 0.0, jnp.abs(gd - pd), 0.0))
        tot = ls[0]
        for b in range(1, 8):
            tot = tot + ls[b]
        lo = [jnp.minimum(ls[2 * i], ls[2 * i + 1]) for i in range(4)]
        hi = [jnp.maximum(ls[2 * i], ls[2 * i + 1]) for i in range(4)]
        m1l = jnp.minimum(lo[0], lo[1])
        m1h = jnp.minimum(jnp.maximum(lo[0], lo[1]),
                          jnp.minimum(hi[0], hi[1]))
        m2l = jnp.minimum(lo[2], lo[3])
        m2h = jnp.minimum(jnp.maximum(lo[2], lo[3]),
                          jnp.minimum(hi[2], hi[3]))
        f1 = jnp.minimum(m1l, m2l)
        f2 = jnp.minimum(jnp.maximum(m1l, m2l), jnp.minimum(m1h, m2h))
        acc = acc + (tot - f1 - f2)
    return acc


def _sc_body(t_hbm, i1_hbm, p2_hbm, out_hbm,
             i1_v, p2i_v, slab_a, slab_b, pb_a, pb_b, row_v, slab16, shared,
             ssa, ssb, spa, spb):
    c = lax.axis_index("c")
    s = lax.axis_index("s")
    wid = s * 2 + c

    pltpu.sync_copy(i1_hbm.at[wid], i1_v)
    pltpu.sync_copy(p2_hbm.at[wid], p2i_v)

    iota = lax.iota(jnp.int32, 16)

    def issue(ci, abuf, pbuf, ssem, psem):
        for d in range(NDESC):
            pltpu.async_copy(t_hbm.at[i1_v.at[ci, d]],
                             abuf.at[pl.ds(d * 128, 128), :], ssem)
            pltpu.async_copy(t_hbm.at[p2i_v.at[ci, d]],
                             pbuf.at[pl.ds(d * 128, 128), :], psem)

    def wait(abuf, pbuf, ssem, psem):
        for d in range(NDESC):
            pltpu.make_async_copy(t_hbm.at[i1_v.at[0, 0]],
                                  abuf.at[pl.ds(d * 128, 128), :],
                                  ssem).wait()
            pltpu.make_async_copy(t_hbm.at[p2i_v.at[0, 0]],
                                  pbuf.at[pl.ds(d * 128, 128), :],
                                  psem).wait()

    issue(0, slab_a, pb_a, ssa, spa)

    def loop(i, acc):
        c0 = 2 * i
        issue(c0 + 1, slab_b, pb_b, ssb, spb)
        wait(slab_a, pb_a, ssa, spa)
        acc = _pair_compute(slab_a, pb_a, acc, iota)

        @pl.when(i < (NPCH // 2) - 1)
        def _():
            issue(c0 + 2, slab_a, pb_a, ssa, spa)

        wait(slab_b, pb_b, ssb, spb)
        acc = _pair_compute(slab_b, pb_b, acc, iota)
        return acc

    acc = lax.fori_loop(0, NPCH // 2, loop, jnp.zeros((16,), jnp.float32))

    row_v[0, :] = acc
    pltpu.sync_copy(row_v, shared.at[pl.ds(s, 1), :])
    plsc.subcore_barrier()

    @pl.when(s == 0)
    def _():
        pltpu.sync_copy(shared, slab16)
        tot = slab16[0, :]
        for r in range(1, 16):
            tot = tot + slab16[r, :]
        row_v[0, :] = tot
        pltpu.sync_copy(row_v, out_hbm.at[c])


_sc_pairloss = functools.partial(
    pl.kernel,
    mesh=_MESH,
    compiler_params=_COMPILER_PARAMS,
    out_type=jax.ShapeDtypeStruct((2, 1, 16), jnp.float32),
    scratch_types=[
        pltpu.VMEM((NPCH, NDESC, 128), jnp.int32),
        pltpu.VMEM((NPCH, NDESC, 128), jnp.int32),
        pltpu.VMEM((PC, 16), jnp.float32),
        pltpu.VMEM((PC, 16), jnp.float32),
        pltpu.VMEM((PC, 16), jnp.float32),
        pltpu.VMEM((PC, 16), jnp.float32),
        pltpu.VMEM((1, 16), jnp.float32),
        pltpu.VMEM((16, 16), jnp.float32),
        pltpu.VMEM_SHARED((16, 16), jnp.float32),
        pltpu.SemaphoreType.DMA,
        pltpu.SemaphoreType.DMA,
        pltpu.SemaphoreType.DMA,
        pltpu.SemaphoreType.DMA,
    ],
)(_sc_body)


def kernel(gt_depth, pred_depth):
    table = _sc_transpose(gt_depth, pred_depth)
    i1 = jnp.asarray(_I1_NP)
    p2 = jnp.asarray(_P2_NP)
    parts = _sc_pairloss(table.reshape(NUM, 16), i1, p2)
    return jnp.sum(parts) * np.float32(1.0 / (6 * NPAIR))


# speedup vs baseline: 1.3046x; 1.0665x over previous
import functools

import jax
import jax.numpy as jnp
import numpy as np
from jax import lax
from jax.experimental import pallas as pl
from jax.experimental.pallas import tpu as pltpu
from jax.experimental.pallas import tpu_sc as plsc

H = W = 512
NUM = H * W
NPAIR = int(NUM * 0.15)
NTILE = 32

TP_CHUNK = 1024
TP_NCHUNK = NUM // NTILE // TP_CHUNK

PCH = 2048
NPCH = 4
NCH = NTILE * NPCH

_COMPILER_PARAMS = pltpu.CompilerParams(
    needs_layout_passes=False, use_tc_tiling_on_sc=False)
_COMPILER_PARAMS_TILED = pltpu.CompilerParams(
    needs_layout_passes=False, use_tc_tiling_on_sc=True)
_MESH = plsc.VectorSubcoreMesh(core_axis_name="c", subcore_axis_name="s")


def _pair_partition():
    rng = np.random.RandomState(0)
    p1 = rng.choice(NUM, NPAIR, replace=True)
    rng.shuffle(p1)
    p2 = rng.choice(NUM, NPAIR, replace=True)
    rng.shuffle(p2)
    counts = np.zeros(NCH, np.int64)
    anchor = np.empty(NPAIR, np.int64)
    other = np.empty(NPAIR, np.int64)
    c1 = p1 // PCH
    c2 = p2 // PCH
    for i in range(NPAIR):
        if counts[c1[i]] <= counts[c2[i]]:
            a, o = p1[i], p2[i]
        else:
            a, o = p2[i], p1[i]
        anchor[i] = a
        other[i] = o
        counts[a // PCH] += 1
    pc = int(-(-counts.max() // 64) * 64)
    i1 = np.zeros((NCH, pc), np.int32)
    p2g = np.zeros((NCH, pc), np.int32)
    for c in range(NCH):
        i1[c, :] = c * PCH
        p2g[c, :] = c * PCH
    fill = np.zeros(NCH, np.int64)
    for i in range(NPAIR):
        c = anchor[i] // PCH
        j = fill[c]
        fill[c] += 1
        i1[c, j] = anchor[i]
        p2g[c, j] = other[i]
    for c in range(NCH):
        order = np.argsort(i1[c], kind="stable")
        i1[c] = i1[c][order]
        p2g[c] = p2g[c][order]
    return (i1.reshape(NTILE, NPCH, pc),
            p2g.reshape(NTILE, NPCH, pc), pc)


_I1_NP, _P2_NP, PC = _pair_partition()
_DCHUNKS = [(o, min(128, PC - o)) for o in range(0, PC, 128)]


def _tp_body(gt_hbm, pr_hbm, t_hbm, slab_a, slab_b, tch_a, tch_b,
             sem_in_a, sem_in_b, sem_out_a, sem_out_b):
    c = lax.axis_index("c")
    s = lax.axis_index("s")
    wid = s * 2 + c
    base_reg = wid * TP_NCHUNK
    iota = lax.iota(jnp.int32, 16)
    lane_hi = lax.shift_right_logical(iota, 3)
    lane_lo16 = (iota & 7) * 16
    lane_idx = [lane_lo16 + k for k in range(16)]

    def issue_slabs(ri, slab, sem):
        y0 = lax.shift_right_logical(ri, 2) * 8
        x0 = (ri & 3) * 128
        for b in range(8):
            pltpu.async_copy(
                gt_hbm.at[b, 0, pl.ds(y0, 8), pl.ds(x0, 128)],
                slab.at[b], sem)
            pltpu.async_copy(
                pr_hbm.at[b, 0, pl.ds(y0, 8), pl.ds(x0, 128)],
                slab.at[b + 8], sem)

    def wait_slabs(slab, sem):
        for k in range(16):
            pltpu.make_async_copy(
                gt_hbm.at[0, 0, pl.ds(0, 8), pl.ds(0, 128)],
                slab.at[k], sem).wait()

    def compute(ri, slab, tch, sem_out):
        def group(g, _):
            dy = lax.shift_right_logical(g, 3)
            xg = g & 7
            dyv = jnp.full((16,), 0, jnp.int32) + dy
            trow = xg * 2 + lane_hi
            for k in range(16):
                v = slab[k, dy, pl.ds(xg * 16, 16)]
                plsc.store_scatter(tch, [dyv, trow, lane_idx[k]], v)
            return 0
        lax.fori_loop(0, 64, group, 0)
        y0 = lax.shift_right_logical(ri, 2) * 8
        x0r = (ri & 3) * 16
        for dy in range(8):
            tr0 = (y0 + dy) * 64 + x0r
            pltpu.async_copy(tch.at[dy], t_hbm.at[pl.ds(tr0, 16), :],
                             sem_out)

    def wait_out(tch, sem):
        for dy in range(8):
            pltpu.make_async_copy(t_hbm.at[pl.ds(0, 16), :], tch.at[dy],
                                  sem).wait()

    issue_slabs(base_reg, slab_a, sem_in_a)

    def loop(i, carry):
        r0 = base_reg + 2 * i
        issue_slabs(r0 + 1, slab_b, sem_in_b)
        wait_slabs(slab_a, sem_in_a)

        @pl.when(i >= 1)
        def _():
            wait_out(tch_a, sem_out_a)

        compute(r0, slab_a, tch_a, sem_out_a)

        @pl.when(i < (TP_NCHUNK // 2) - 1)
        def _():
            issue_slabs(r0 + 2, slab_a, sem_in_a)

        wait_slabs(slab_b, sem_in_b)

        @pl.when(i >= 1)
        def _():
            wait_out(tch_b, sem_out_b)

        compute(r0 + 1, slab_b, tch_b, sem_out_b)
        return carry

    lax.fori_loop(0, TP_NCHUNK // 2, loop, 0)
    wait_out(tch_a, sem_out_a)
    wait_out(tch_b, sem_out_b)


_sc_transpose = functools.partial(
    pl.kernel,
    mesh=_MESH,
    compiler_params=_COMPILER_PARAMS_TILED,
    out_type=jax.ShapeDtypeStruct((NUM // 8, 128), jnp.float32),
    scratch_types=[
        pltpu.VMEM((16, 8, 128), jnp.float32),
        pltpu.VMEM((16, 8, 128), jnp.float32),
        pltpu.VMEM((8, 16, 128), jnp.float32),
        pltpu.VMEM((8, 16, 128), jnp.float32),
        pltpu.SemaphoreType.DMA,
        pltpu.SemaphoreType.DMA,
        pltpu.SemaphoreType.DMA,
        pltpu.SemaphoreType.DMA,
    ],
)(_tp_body)


def _pair_compute(abuf, pbuf, acc, iota):
    for g in range(PC // 16):
        rowi = g * 16 + iota
        ls = []
        for b in range(8):
            cb = jnp.full((16,), b, jnp.int32)
            cq = jnp.full((16,), b + 8, jnp.int32)
            g1 = plsc.load_gather(abuf, [rowi, cb])
            g2 = plsc.load_gather(pbuf, [rowi, cb])
            q1 = plsc.load_gather(abuf, [rowi, cq])
            q2 = plsc.load_gather(pbuf, [rowi, cq])
            gd = g1 - g2
            pd = q1 - q2
            ls.append(jnp.where(gd - gd == 0.0, jnp.abs(gd - pd), 0.0))
        tot = ls[0]
        for b in range(1, 8):
            tot = tot + ls[b]
        lo = [jnp.minimum(ls[2 * i], ls[2 * i + 1]) for i in range(4)]
        hi = [jnp.maximum(ls[2 * i], ls[2 * i + 1]) for i in range(4)]
        m1l = jnp.minimum(lo[0], lo[1])
        m1h = jnp.minimum(jnp.maximum(lo[0], lo[1]),
                          jnp.minimum(hi[0], hi[1]))
        m2l = jnp.minimum(lo[2], lo[3])
        m2h = jnp.minimum(jnp.maximum(lo[2], lo[3]),
                          jnp.minimum(hi[2], hi[3]))
        f1 = jnp.minimum(m1l, m2l)
        f2 = jnp.minimum(jnp.maximum(m1l, m2l), jnp.minimum(m1h, m2h))
        acc = acc + (tot - f1 - f2)
    return acc


def _sc_body(t_hbm, i1_hbm, p2_hbm, out_hbm,
             i1_v, p2i_v, slab_a, slab_b, pb_a, pb_b, row_v, slab16, shared,
             ssa, ssb, spa, spb):
    c = lax.axis_index("c")
    s = lax.axis_index("s")
    wid = s * 2 + c

    pltpu.sync_copy(i1_hbm.at[wid], i1_v)
    pltpu.sync_copy(p2_hbm.at[wid], p2i_v)

    iota = lax.iota(jnp.int32, 16)

    def issue(ci, abuf, pbuf, ssem, psem):
        for o, w in _DCHUNKS:
            pltpu.async_copy(t_hbm.at[i1_v.at[ci, pl.ds(o, w)]],
                             abuf.at[pl.ds(o, w), :], ssem)
            pltpu.async_copy(t_hbm.at[p2i_v.at[ci, pl.ds(o, w)]],
                             pbuf.at[pl.ds(o, w), :], psem)

    def wait(abuf, pbuf, ssem, psem):
        for o, w in _DCHUNKS:
            pltpu.make_async_copy(t_hbm.at[i1_v.at[0, pl.ds(0, w)]],
                                  abuf.at[pl.ds(o, w), :],
                                  ssem).wait()
            pltpu.make_async_copy(t_hbm.at[p2i_v.at[0, pl.ds(0, w)]],
                                  pbuf.at[pl.ds(o, w), :],
                                  psem).wait()

    issue(0, slab_a, pb_a, ssa, spa)

    def loop(i, acc):
        c0 = 2 * i
        issue(c0 + 1, slab_b, pb_b, ssb, spb)
        wait(slab_a, pb_a, ssa, spa)
        acc = _pair_compute(slab_a, pb_a, acc, iota)

        @pl.when(i < (NPCH // 2) - 1)
        def _():
            issue(c0 + 2, slab_a, pb_a, ssa, spa)

        wait(slab_b, pb_b, ssb, spb)
        acc = _pair_compute(slab_b, pb_b, acc, iota)
        return acc

    acc = lax.fori_loop(0, NPCH // 2, loop, jnp.zeros((16,), jnp.float32))

    row_v[0, :] = acc
    pltpu.sync_copy(row_v, shared.at[pl.ds(s, 1), :])
    plsc.subcore_barrier()

    @pl.when(s == 0)
    def _():
        pltpu.sync_copy(shared, slab16)
        tot = slab16[0, :]
        for r in range(1, 16):
            tot = tot + slab16[r, :]
        row_v[0, :] = tot
        pltpu.sync_copy(row_v, out_hbm.at[c])


_sc_pairloss = functools.partial(
    pl.kernel,
    mesh=_MESH,
    compiler_params=_COMPILER_PARAMS,
    out_type=jax.ShapeDtypeStruct((2, 1, 16), jnp.float32),
    scratch_types=[
        pltpu.VMEM((NPCH, PC), jnp.int32),
        pltpu.VMEM((NPCH, PC), jnp.int32),
        pltpu.VMEM((PC, 16), jnp.float32),
        pltpu.VMEM((PC, 16), jnp.float32),
        pltpu.VMEM((PC, 16), jnp.float32),
        pltpu.VMEM((PC, 16), jnp.float32),
        pltpu.VMEM((1, 16), jnp.float32),
        pltpu.VMEM((16, 16), jnp.float32),
        pltpu.VMEM_SHARED((16, 16), jnp.float32),
        pltpu.SemaphoreType.DMA,
        pltpu.SemaphoreType.DMA,
        pltpu.SemaphoreType.DMA,
        pltpu.SemaphoreType.DMA,
    ],
)(_sc_body)


def kernel(gt_depth, pred_depth):
    table = _sc_transpose(gt_depth, pred_depth)
    i1 = jnp.asarray(_I1_NP)
    p2 = jnp.asarray(_P2_NP)
    parts = _sc_pairloss(table.reshape(NUM, 16), i1, p2)
    return jnp.sum(parts) * np.float32(1.0 / (6 * NPAIR))
